# baseline (speedup 1.0000x reference)
import functools

import jax
import jax.numpy as jnp
from jax import lax
from jax.experimental import pallas as pl
from jax.experimental.pallas import tpu as pltpu
from jax.experimental.pallas import tpu_sc as plsc

N = 1000000
RHO = 0.99
ALPHA = 0.5
TEMP = 0.5
B = 4096
EPS = 1e-6

NC = 2
NS = 16
NW = NC * NS
CHUNK = B // NW
LANES = 16
NCOPY = 4
CC = N // NCOPY


def _sc_update_body(s_ref, idx_hbm, vr_hbm, rep_hbm,
                    idx_v, s_v, v4_v, r2_v, out_v, rep_v, sem1, sem2, sem3):
    wid = lax.axis_index("s") * NC + lax.axis_index("c")
    base = wid * CHUNK
    ci = pltpu.async_copy(idx_hbm.at[pl.ds(base, CHUNK)], idx_v, sem1)
    cv = pltpu.async_copy(vr_hbm.at[pl.ds(base, CHUNK)], v4_v, sem2)
    cr = pltpu.async_copy(vr_hbm.at[pl.ds(B + base, CHUNK)], r2_v, sem3)
    ci.wait()
    pltpu.async_copy(s_ref.at[idx_v], s_v, sem1).wait()
    cv.wait()
    cr.wait()
    scale = jnp.float32((1.0 - RHO) * float(N) * float(N))
    rep_acc = jnp.zeros((LANES,), jnp.float32)
    for j in range(CHUNK // LANES):
        sl = pl.ds(j * LANES, LANES)
        s = s_v[sl]
        out_v[sl] = RHO * s + scale * v4_v[sl]
        rep_acc = rep_acc + r2_v[sl] / s
    rep_v[...] = rep_acc
    cs = pltpu.async_copy(out_v, s_ref.at[idx_v], sem1)
    cp = pltpu.async_copy(rep_v, rep_hbm.at[pl.ds(wid * LANES, LANES)], sem2)
    cs.wait()
    cp.wait()


def _dense_body(f_hbm, s_inv_hbm, s_ref, vr_hbm, f_v, vr_v, semf, semc, semo):
    cf = pltpu.async_copy(f_hbm, f_v, semf)
    copies = [pltpu.async_copy(s_inv_hbm, s_ref, semc)]
    cf.wait()
    f = f_v[...]
    norm = jnp.maximum(jnp.sqrt(jnp.sum(f * f, axis=1, keepdims=True)), 1e-12)
    fn = f / norm
    an = fn[:B]
    bn = fn[B:]
    bro = pltpu.roll(bn, B - 1, 0)
    aro = pltpu.roll(an, B - 1, 0)
    d2aa = jnp.sum((an - bn + EPS) ** 2, axis=1, keepdims=True)
    d2bb = jnp.sum((bn - an + EPS) ** 2, axis=1, keepdims=True)
    d2ra = jnp.sum((an - bro + EPS) ** 2, axis=1, keepdims=True)
    d2rb = jnp.sum((bn - aro + EPS) ** 2, axis=1, keepdims=True)
    inv2t2 = 1.0 / (2.0 * TEMP * TEMP)
    qaa = jnp.exp(-inv2t2 * d2aa)
    qab = jnp.exp(-inv2t2 * d2bb)
    qra = jnp.exp(-inv2t2 * d2ra)
    qrb = jnp.exp(-inv2t2 * d2rb)
    v4 = (ALPHA * 0.5) * (qaa + qab) + ((1.0 - ALPHA) * 0.5) * (qra + qrb)
    r2 = qra + qrb
    attr = inv2t2 * jnp.sum(d2aa + d2bb)
    vr_v[0:32, :] = v4.reshape(32, 128)
    vr_v[32:64, :] = r2.reshape(32, 128)
    vr_v[64:72, :] = jnp.full((8, 128), attr, jnp.float32)
    co = pltpu.async_copy(vr_v, vr_hbm, semo)
    co.wait()
    for c in copies:
        c.wait()


@functools.cache
def _build():
    sc_mesh = plsc.VectorSubcoreMesh(
        core_axis_name="c", subcore_axis_name="s", num_cores=NC, num_subcores=NS
    )
    sc_update = pl.kernel(
        _sc_update_body,
        out_type=jax.ShapeDtypeStruct((NW * LANES,), jnp.float32),
        mesh=sc_mesh,
        scratch_types=[
            pltpu.VMEM((CHUNK,), jnp.int32),
            pltpu.VMEM((CHUNK,), jnp.float32),
            pltpu.VMEM((CHUNK,), jnp.float32),
            pltpu.VMEM((CHUNK,), jnp.float32),
            pltpu.VMEM((CHUNK,), jnp.float32),
            pltpu.VMEM((LANES,), jnp.float32),
            pltpu.SemaphoreType.DMA,
            pltpu.SemaphoreType.DMA,
            pltpu.SemaphoreType.DMA,
        ],
    )
    tc_mesh = pltpu.create_tensorcore_mesh("t", num_cores=1)
    dense = pl.kernel(
        _dense_body,
        out_type=jax.ShapeDtypeStruct((72, 128), jnp.float32),
        mesh=tc_mesh,
        scratch_types=[
            pltpu.VMEM((2 * B, 128), jnp.float32),
            pltpu.VMEM((72, 128), jnp.float32),
            pltpu.SemaphoreType.DMA,
            pltpu.SemaphoreType.DMA,
            pltpu.SemaphoreType.DMA,
        ],
    )
    return sc_update, dense


def kernel(feats, s_inv, feats_idx):
    sc_update, dense = _build()
    idx = feats_idx.astype(jnp.int32)
    s_ref = jax.new_ref(jnp.zeros((N,), jnp.float32))
    vr = dense(feats, s_inv, s_ref)
    flat = vr.reshape(72 * 128)
    rep = sc_update(s_ref, idx, flat)
    new_s_inv = s_ref[...]
    n2 = jnp.float32(N) * jnp.float32(N)
    loss = 0.5 * (flat[2 * B] + n2 * jnp.sum(rep)) / jnp.float32(B)
    return loss, new_s_inv


# speedup vs baseline: 3.9664x; 3.9664x over previous
import functools

import jax
import jax.numpy as jnp
from jax import lax
from jax.experimental import pallas as pl
from jax.experimental.pallas import tpu as pltpu
from jax.experimental.pallas import tpu_sc as plsc

N = 1000000
RHO = 0.99
ALPHA = 0.5
TEMP = 0.5
B = 4096
EPS = 1e-6

NC = 2
NS = 16
NW = NC * NS
CHUNK = B // NW
LANES = 16


def _sc_gather_body(s_inv_hbm, idx_hbm, out_hbm, idx_v, s_v, sem):
    wid = lax.axis_index("s") * NC + lax.axis_index("c")
    base = wid * CHUNK
    pltpu.async_copy(idx_hbm.at[pl.ds(base, CHUNK)], idx_v, sem).wait()
    pltpu.async_copy(s_inv_hbm.at[idx_v], s_v, sem).wait()
    pltpu.async_copy(s_v, out_hbm.at[pl.ds(base, CHUNK)], sem).wait()


def _sc_update_body(s_ref, idx_hbm, sg_hbm, vr_hbm, rep_hbm,
                    idx_v, s_v, v4_v, r2_v, out_v, rep_v,
                    sem1, sem2, sem3, sem4):
    wid = lax.axis_index("s") * NC + lax.axis_index("c")
    base = wid * CHUNK
    ci = pltpu.async_copy(idx_hbm.at[pl.ds(base, CHUNK)], idx_v, sem1)
    cg = pltpu.async_copy(sg_hbm.at[pl.ds(base, CHUNK)], s_v, sem2)
    cv = pltpu.async_copy(vr_hbm.at[pl.ds(base, CHUNK)], v4_v, sem3)
    cr = pltpu.async_copy(vr_hbm.at[pl.ds(B + base, CHUNK)], r2_v, sem4)
    cg.wait()
    cv.wait()
    cr.wait()
    scale = jnp.float32((1.0 - RHO) * float(N) * float(N))
    rep_acc = jnp.zeros((LANES,), jnp.float32)
    for j in range(CHUNK // LANES):
        sl = pl.ds(j * LANES, LANES)
        s = s_v[sl]
        out_v[sl] = RHO * s + scale * v4_v[sl]
        rep_acc = rep_acc + r2_v[sl] / s
    rep_v[...] = rep_acc
    ci.wait()
    cs = pltpu.async_copy(out_v, s_ref.at[idx_v], sem1)
    cp = pltpu.async_copy(rep_v, rep_hbm.at[pl.ds(wid * LANES, LANES)], sem2)
    cs.wait()
    cp.wait()


def _dense_body(f_hbm, vr_hbm, f_v, vr_v, semf, semo):
    pltpu.async_copy(f_hbm, f_v, semf).wait()
    f = f_v[...]
    norm = jnp.maximum(jnp.sqrt(jnp.sum(f * f, axis=1, keepdims=True)), 1e-12)
    fn = f / norm
    an = fn[:B]
    bn = fn[B:]
    bro = pltpu.roll(bn, B - 1, 0)
    aro = pltpu.roll(an, B - 1, 0)
    d2aa = jnp.sum((an - bn + EPS) ** 2, axis=1, keepdims=True)
    d2bb = jnp.sum((bn - an + EPS) ** 2, axis=1, keepdims=True)
    d2ra = jnp.sum((an - bro + EPS) ** 2, axis=1, keepdims=True)
    d2rb = jnp.sum((bn - aro + EPS) ** 2, axis=1, keepdims=True)
    inv2t2 = 1.0 / (2.0 * TEMP * TEMP)
    qaa = jnp.exp(-inv2t2 * d2aa)
    qab = jnp.exp(-inv2t2 * d2bb)
    qra = jnp.exp(-inv2t2 * d2ra)
    qrb = jnp.exp(-inv2t2 * d2rb)
    v4 = (ALPHA * 0.5) * (qaa + qab) + ((1.0 - ALPHA) * 0.5) * (qra + qrb)
    r2 = qra + qrb
    attr = inv2t2 * jnp.sum(d2aa + d2bb)
    vr_v[0:32, :] = v4.reshape(32, 128)
    vr_v[32:64, :] = r2.reshape(32, 128)
    vr_v[64:72, :] = jnp.full((8, 128), attr, jnp.float32)
    pltpu.async_copy(vr_v, vr_hbm, semo).wait()


@functools.cache
def _build():
    sc_mesh = plsc.VectorSubcoreMesh(
        core_axis_name="c", subcore_axis_name="s", num_cores=NC, num_subcores=NS
    )
    sc_gather = pl.kernel(
        _sc_gather_body,
        out_type=jax.ShapeDtypeStruct((B,), jnp.float32),
        mesh=sc_mesh,
        scratch_types=[
            pltpu.VMEM((CHUNK,), jnp.int32),
            pltpu.VMEM((CHUNK,), jnp.float32),
            pltpu.SemaphoreType.DMA,
        ],
    )
    sc_update = pl.kernel(
        _sc_update_body,
        out_type=jax.ShapeDtypeStruct((NW * LANES,), jnp.float32),
        mesh=sc_mesh,
        scratch_types=[
            pltpu.VMEM((CHUNK,), jnp.int32),
            pltpu.VMEM((CHUNK,), jnp.float32),
            pltpu.VMEM((CHUNK,), jnp.float32),
            pltpu.VMEM((CHUNK,), jnp.float32),
            pltpu.VMEM((CHUNK,), jnp.float32),
            pltpu.VMEM((LANES,), jnp.float32),
            pltpu.SemaphoreType.DMA,
            pltpu.SemaphoreType.DMA,
            pltpu.SemaphoreType.DMA,
            pltpu.SemaphoreType.DMA,
        ],
    )
    tc_mesh = pltpu.create_tensorcore_mesh("t", num_cores=1)
    dense = pl.kernel(
        _dense_body,
        out_type=jax.ShapeDtypeStruct((72, 128), jnp.float32),
        mesh=tc_mesh,
        scratch_types=[
            pltpu.VMEM((2 * B, 128), jnp.float32),
            pltpu.VMEM((72, 128), jnp.float32),
            pltpu.SemaphoreType.DMA,
            pltpu.SemaphoreType.DMA,
        ],
    )
    return sc_gather, sc_update, dense


def kernel(feats, s_inv, feats_idx):
    sc_gather, sc_update, dense = _build()
    idx = feats_idx.astype(jnp.int32)
    s_gather = sc_gather(s_inv, idx)
    vr = dense(feats)
    s_ref = jax.new_ref(s_inv)
    flat = vr.reshape(72 * 128)
    rep = sc_update(s_ref, idx, s_gather, flat)
    new_s_inv = s_ref[...]
    n2 = jnp.float32(N) * jnp.float32(N)
    loss = 0.5 * (flat[2 * B] + n2 * jnp.sum(rep)) / jnp.float32(B)
    return loss, new_s_inv


# speedup vs baseline: 4.0558x; 1.0226x over previous
import functools

import jax
import jax.numpy as jnp
from jax import lax
from jax.experimental import pallas as pl
from jax.experimental.pallas import tpu as pltpu
from jax.experimental.pallas import tpu_sc as plsc

N = 1000000
RHO = 0.99
ALPHA = 0.5
TEMP = 0.5
B = 4096
EPS = 1e-6

NC = 2
NS = 16
NW = NC * NS
CHUNK = B // NW
LANES = 16
CPCH = 31248
TAIL = N - NW * CPCH
TAIL_OFF = NW * CPCH


def _sc_gather_body(s_inv_hbm, idx_hbm, s_ref, sg_hbm,
                    idx_v, s_v, buf_v, tail_v, semi, semg, semc):
    wid = lax.axis_index("s") * NC + lax.axis_index("c")
    base = wid * CHUNK
    lo = wid * CPCH
    ci = pltpu.async_copy(idx_hbm.at[pl.ds(base, CHUNK)], idx_v, semi)
    cc = pltpu.async_copy(s_inv_hbm.at[pl.ds(lo, CPCH)], buf_v, semc)
    ct = pltpu.async_copy(s_inv_hbm.at[pl.ds(TAIL_OFF, TAIL)], tail_v, semg)
    ci.wait()
    cg = pltpu.async_copy(s_inv_hbm.at[idx_v], s_v, semi)
    cc.wait()
    cc2 = pltpu.async_copy(buf_v, s_ref.at[pl.ds(lo, CPCH)], semc)
    ct.wait()
    ct2 = pltpu.async_copy(tail_v, s_ref.at[pl.ds(TAIL_OFF, TAIL)], semg)
    cg.wait()
    co = pltpu.async_copy(s_v, sg_hbm.at[pl.ds(base, CHUNK)], semi)
    cc2.wait()
    ct2.wait()
    co.wait()


def _sc_update_body(s_ref, idx_hbm, sg_hbm, vr_hbm, rep_hbm,
                    idx_v, s_v, v4_v, r2_v, at_v, out_v, rep_v,
                    sem1, sem2, sem3, sem4):
    wid = lax.axis_index("s") * NC + lax.axis_index("c")
    base = wid * CHUNK
    ci = pltpu.async_copy(idx_hbm.at[pl.ds(base, CHUNK)], idx_v, sem1)
    cg = pltpu.async_copy(sg_hbm.at[pl.ds(base, CHUNK)], s_v, sem2)
    cv = pltpu.async_copy(vr_hbm.at[pl.ds(base, CHUNK)], v4_v, sem3)
    cr = pltpu.async_copy(vr_hbm.at[pl.ds(B + base, CHUNK)], r2_v, sem4)
    cg.wait()
    cv.wait()
    cr.wait()
    scale = jnp.float32((1.0 - RHO) * float(N) * float(N))
    rep_acc = jnp.zeros((LANES,), jnp.float32)
    for j in range(CHUNK // LANES):
        sl = pl.ds(j * LANES, LANES)
        s = s_v[sl]
        out_v[sl] = RHO * s + scale * v4_v[sl]
        rep_acc = rep_acc + r2_v[sl] / s
    ca = pltpu.async_copy(vr_hbm.at[pl.ds(2 * B, LANES)], at_v, sem2)
    ca.wait()
    inv_n2 = 1.0 / (float(N) * float(N) * float(LANES))
    w0 = jnp.where(wid == 0, jnp.float32(inv_n2), jnp.float32(0.0))
    rep_v[...] = rep_acc + at_v[...] * w0
    ci.wait()
    cs = pltpu.async_copy(out_v, s_ref.at[idx_v], sem1)
    cp = pltpu.async_copy(rep_v, rep_hbm.at[pl.ds(wid * LANES, LANES)], sem2)
    cs.wait()
    cp.wait()


def _dense_body(f_hbm, vr_hbm, f_v, vr_v, semf, semo):
    pltpu.async_copy(f_hbm, f_v, semf).wait()
    f = f_v[...]
    norm = jnp.maximum(jnp.sqrt(jnp.sum(f * f, axis=1, keepdims=True)), 1e-12)
    fn = f / norm
    an = fn[:B]
    bn = fn[B:]
    bro = pltpu.roll(bn, B - 1, 0)
    aro = pltpu.roll(an, B - 1, 0)
    d2aa = jnp.sum((an - bn + EPS) ** 2, axis=1, keepdims=True)
    d2bb = jnp.sum((bn - an + EPS) ** 2, axis=1, keepdims=True)
    d2ra = jnp.sum((an - bro + EPS) ** 2, axis=1, keepdims=True)
    d2rb = jnp.sum((bn - aro + EPS) ** 2, axis=1, keepdims=True)
    inv2t2 = 1.0 / (2.0 * TEMP * TEMP)
    qaa = jnp.exp(-inv2t2 * d2aa)
    qab = jnp.exp(-inv2t2 * d2bb)
    qra = jnp.exp(-inv2t2 * d2ra)
    qrb = jnp.exp(-inv2t2 * d2rb)
    v4 = (ALPHA * 0.5) * (qaa + qab) + ((1.0 - ALPHA) * 0.5) * (qra + qrb)
    r2 = qra + qrb
    attr = inv2t2 * jnp.sum(d2aa + d2bb)
    vr_v[0:32, :] = v4.reshape(32, 128)
    vr_v[32:64, :] = r2.reshape(32, 128)
    vr_v[64:72, :] = jnp.full((8, 128), attr, jnp.float32)
    pltpu.async_copy(vr_v, vr_hbm, semo).wait()


@functools.cache
def _build():
    sc_mesh = plsc.VectorSubcoreMesh(
        core_axis_name="c", subcore_axis_name="s", num_cores=NC, num_subcores=NS
    )
    sc_gather = pl.kernel(
        _sc_gather_body,
        out_type=jax.ShapeDtypeStruct((B,), jnp.float32),
        mesh=sc_mesh,
        scratch_types=[
            pltpu.VMEM((CHUNK,), jnp.int32),
            pltpu.VMEM((CHUNK,), jnp.float32),
            pltpu.VMEM((CPCH,), jnp.float32),
            pltpu.VMEM((TAIL,), jnp.float32),
            pltpu.SemaphoreType.DMA,
            pltpu.SemaphoreType.DMA,
            pltpu.SemaphoreType.DMA,
        ],
    )
    sc_update = pl.kernel(
        _sc_update_body,
        out_type=jax.ShapeDtypeStruct((NW * LANES,), jnp.float32),
        mesh=sc_mesh,
        scratch_types=[
            pltpu.VMEM((CHUNK,), jnp.int32),
            pltpu.VMEM((CHUNK,), jnp.float32),
            pltpu.VMEM((CHUNK,), jnp.float32),
            pltpu.VMEM((CHUNK,), jnp.float32),
            pltpu.VMEM((LANES,), jnp.float32),
            pltpu.VMEM((CHUNK,), jnp.float32),
            pltpu.VMEM((LANES,), jnp.float32),
            pltpu.SemaphoreType.DMA,
            pltpu.SemaphoreType.DMA,
            pltpu.SemaphoreType.DMA,
            pltpu.SemaphoreType.DMA,
        ],
    )
    tc_mesh = pltpu.create_tensorcore_mesh("t", num_cores=1)
    dense = pl.kernel(
        _dense_body,
        out_type=jax.ShapeDtypeStruct((72, 128), jnp.float32),
        mesh=tc_mesh,
        scratch_types=[
            pltpu.VMEM((2 * B, 128), jnp.float32),
            pltpu.VMEM((72, 128), jnp.float32),
            pltpu.SemaphoreType.DMA,
            pltpu.SemaphoreType.DMA,
        ],
    )
    return sc_gather, sc_update, dense


def kernel(feats, s_inv, feats_idx):
    sc_gather, sc_update, dense = _build()
    idx = feats_idx.astype(jnp.int32)
    s_ref = jax.new_ref(jnp.zeros((N,), jnp.float32))
    s_gather = sc_gather(s_inv, idx, s_ref)
    vr = dense(feats)
    flat = vr.reshape(72 * 128)
    rep = sc_update(s_ref, idx, s_gather, flat)
    new_s_inv = s_ref[...]
    n2 = jnp.float32(N) * jnp.float32(N)
    loss = 0.5 * n2 * jnp.sum(rep) / jnp.float32(B)
    return loss, new_s_inv


# speedup vs baseline: 4.0750x; 1.0047x over previous
import functools

import jax
import jax.numpy as jnp
from jax import lax
from jax.experimental import pallas as pl
from jax.experimental.pallas import tpu as pltpu
from jax.experimental.pallas import tpu_sc as plsc

N = 1000000
RHO = 0.99
ALPHA = 0.5
TEMP = 0.5
B = 4096
EPS = 1e-6

NC = 2
NS = 16
NW = NC * NS
CHUNK = B // NW
LANES = 16
CPCH = 31248
TAIL = N - NW * CPCH
TAIL_OFF = NW * CPCH


def _sc_gather_body(s_inv_hbm, idx_hbm, s_ref, sg_hbm,
                    idx_v, s_v, buf_v, tail_v, semi, semg, semc):
    wid = lax.axis_index("s") * NC + lax.axis_index("c")
    base = wid * CHUNK
    lo = wid * CPCH
    ci = pltpu.async_copy(idx_hbm.at[pl.ds(base, CHUNK)], idx_v, semi)
    cc = pltpu.async_copy(s_inv_hbm.at[pl.ds(lo, CPCH)], buf_v, semc)
    ct = pltpu.async_copy(s_inv_hbm.at[pl.ds(TAIL_OFF, TAIL)], tail_v, semg)
    ci.wait()
    cg = pltpu.async_copy(s_inv_hbm.at[idx_v], s_v, semi)
    cc.wait()
    cc2 = pltpu.async_copy(buf_v, s_ref.at[pl.ds(lo, CPCH)], semc)
    ct.wait()
    ct2 = pltpu.async_copy(tail_v, s_ref.at[pl.ds(TAIL_OFF, TAIL)], semg)
    cg.wait()
    co = pltpu.async_copy(s_v, sg_hbm.at[pl.ds(base, CHUNK)], semi)
    cc2.wait()
    ct2.wait()
    co.wait()


def _sc_update_body(s_ref, idx_hbm, sg_hbm, vr_hbm, rep_hbm,
                    idx_v, s_v, v4_v, r2_v, at_v, out_v, rep_v,
                    sem1, sem2, sem3, sem4, sem5):
    wid = lax.axis_index("s") * NC + lax.axis_index("c")
    base = wid * CHUNK
    ci = pltpu.async_copy(idx_hbm.at[pl.ds(base, CHUNK)], idx_v, sem1)
    cg = pltpu.async_copy(sg_hbm.at[pl.ds(base, CHUNK)], s_v, sem2)
    cv = pltpu.async_copy(vr_hbm.at[pl.ds(base, CHUNK)], v4_v, sem3)
    cr = pltpu.async_copy(vr_hbm.at[pl.ds(B + base, CHUNK)], r2_v, sem4)
    ca = pltpu.async_copy(vr_hbm.at[pl.ds(2 * B, LANES)], at_v, sem5)
    cg.wait()
    cv.wait()
    cr.wait()
    scale = jnp.float32((1.0 - RHO) * float(N) * float(N))
    rep_acc = jnp.zeros((LANES,), jnp.float32)
    for j in range(CHUNK // LANES):
        sl = pl.ds(j * LANES, LANES)
        s = s_v[sl]
        out_v[sl] = RHO * s + scale * v4_v[sl]
        rep_acc = rep_acc + r2_v[sl] / s
    ca.wait()
    inv_n2 = 1.0 / (float(N) * float(N) * float(LANES))
    w0 = jnp.where(wid == 0, jnp.float32(inv_n2), jnp.float32(0.0))
    rep_v[...] = rep_acc + at_v[...] * w0
    ci.wait()
    cs = pltpu.async_copy(out_v, s_ref.at[idx_v], sem1)
    cp = pltpu.async_copy(rep_v, rep_hbm.at[pl.ds(wid * LANES, LANES)], sem2)
    cs.wait()
    cp.wait()


def _dense_body(f_hbm, vr_hbm, f_v, vr_v, semf, semo):
    pltpu.async_copy(f_hbm, f_v, semf).wait()
    f = f_v[...]
    norm = jnp.maximum(jnp.sqrt(jnp.sum(f * f, axis=1, keepdims=True)), 1e-12)
    fn = f / norm
    an = fn[:B]
    bn = fn[B:]
    bro = pltpu.roll(bn, B - 1, 0)
    aro = pltpu.roll(an, B - 1, 0)
    d2aa = jnp.sum((an - bn + EPS) ** 2, axis=1, keepdims=True)
    d2bb = jnp.sum((bn - an + EPS) ** 2, axis=1, keepdims=True)
    d2ra = jnp.sum((an - bro + EPS) ** 2, axis=1, keepdims=True)
    d2rb = jnp.sum((bn - aro + EPS) ** 2, axis=1, keepdims=True)
    inv2t2 = 1.0 / (2.0 * TEMP * TEMP)
    qaa = jnp.exp(-inv2t2 * d2aa)
    qab = jnp.exp(-inv2t2 * d2bb)
    qra = jnp.exp(-inv2t2 * d2ra)
    qrb = jnp.exp(-inv2t2 * d2rb)
    v4 = (ALPHA * 0.5) * (qaa + qab) + ((1.0 - ALPHA) * 0.5) * (qra + qrb)
    r2 = qra + qrb
    attr = inv2t2 * jnp.sum(d2aa + d2bb)
    vr_v[0:32, :] = v4.reshape(32, 128)
    vr_v[32:64, :] = r2.reshape(32, 128)
    vr_v[64:72, :] = jnp.full((8, 128), attr, jnp.float32)
    pltpu.async_copy(vr_v, vr_hbm, semo).wait()


@functools.cache
def _build():
    sc_mesh = plsc.VectorSubcoreMesh(
        core_axis_name="c", subcore_axis_name="s", num_cores=NC, num_subcores=NS
    )
    sc_gather = pl.kernel(
        _sc_gather_body,
        out_type=jax.ShapeDtypeStruct((B,), jnp.float32),
        mesh=sc_mesh,
        scratch_types=[
            pltpu.VMEM((CHUNK,), jnp.int32),
            pltpu.VMEM((CHUNK,), jnp.float32),
            pltpu.VMEM((CPCH,), jnp.float32),
            pltpu.VMEM((TAIL,), jnp.float32),
            pltpu.SemaphoreType.DMA,
            pltpu.SemaphoreType.DMA,
            pltpu.SemaphoreType.DMA,
        ],
    )
    sc_update = pl.kernel(
        _sc_update_body,
        out_type=jax.ShapeDtypeStruct((NW * LANES,), jnp.float32),
        mesh=sc_mesh,
        scratch_types=[
            pltpu.VMEM((CHUNK,), jnp.int32),
            pltpu.VMEM((CHUNK,), jnp.float32),
            pltpu.VMEM((CHUNK,), jnp.float32),
            pltpu.VMEM((CHUNK,), jnp.float32),
            pltpu.VMEM((LANES,), jnp.float32),
            pltpu.VMEM((CHUNK,), jnp.float32),
            pltpu.VMEM((LANES,), jnp.float32),
            pltpu.SemaphoreType.DMA,
            pltpu.SemaphoreType.DMA,
            pltpu.SemaphoreType.DMA,
            pltpu.SemaphoreType.DMA,
            pltpu.SemaphoreType.DMA,
        ],
    )
    tc_mesh = pltpu.create_tensorcore_mesh("t", num_cores=1)
    dense = pl.kernel(
        _dense_body,
        out_type=jax.ShapeDtypeStruct((72, 128), jnp.float32),
        mesh=tc_mesh,
        scratch_types=[
            pltpu.VMEM((2 * B, 128), jnp.float32),
            pltpu.VMEM((72, 128), jnp.float32),
            pltpu.SemaphoreType.DMA,
            pltpu.SemaphoreType.DMA,
        ],
    )
    return sc_gather, sc_update, dense


def kernel(feats, s_inv, feats_idx):
    sc_gather, sc_update, dense = _build()
    idx = feats_idx.astype(jnp.int32)
    s_ref = jax.new_ref(jnp.zeros((N,), jnp.float32))
    s_gather = sc_gather(s_inv, idx, s_ref)
    vr = dense(feats)
    flat = vr.reshape(72 * 128)
    rep = sc_update(s_ref, idx, s_gather, flat)
    new_s_inv = s_ref[...]
    n2 = jnp.float32(N) * jnp.float32(N)
    loss = 0.5 * n2 * jnp.sum(rep) / jnp.float32(B)
    return loss, new_s_inv


# speedup vs baseline: 4.1450x; 1.0172x over previous
import functools

import jax
import jax.numpy as jnp
from jax import lax
from jax._src.core import empty_ref as _empty_ref
from jax.experimental import pallas as pl
from jax.experimental.pallas import tpu as pltpu
from jax.experimental.pallas import tpu_sc as plsc

N = 1000000
RHO = 0.99
ALPHA = 0.5
TEMP = 0.5
B = 4096
EPS = 1e-6

NC = 2
NS = 16
NW = NC * NS
CHUNK = B // NW
LANES = 16
CPCH = 31248
TAIL = N - NW * CPCH
TAIL_OFF = NW * CPCH


def _sc_gather_body(s_inv_hbm, idx_hbm, s_ref, sg_hbm,
                    idx_v, s_v, buf_v, tail_v, semi, semg, semc):
    wid = lax.axis_index("s") * NC + lax.axis_index("c")
    base = wid * CHUNK
    lo = wid * CPCH
    ci = pltpu.async_copy(idx_hbm.at[pl.ds(base, CHUNK)], idx_v, semi)
    cc = pltpu.async_copy(s_inv_hbm.at[pl.ds(lo, CPCH)], buf_v, semc)
    ct = pltpu.async_copy(s_inv_hbm.at[pl.ds(TAIL_OFF, TAIL)], tail_v, semg)
    ci.wait()
    cg = pltpu.async_copy(s_inv_hbm.at[idx_v], s_v, semi)
    cc.wait()
    cc2 = pltpu.async_copy(buf_v, s_ref.at[pl.ds(lo, CPCH)], semc)
    ct.wait()
    ct2 = pltpu.async_copy(tail_v, s_ref.at[pl.ds(TAIL_OFF, TAIL)], semg)
    cg.wait()
    co = pltpu.async_copy(s_v, sg_hbm.at[pl.ds(base, CHUNK)], semi)
    cc2.wait()
    ct2.wait()
    co.wait()


def _sc_update_body(s_ref, idx_hbm, sg_hbm, vr_hbm, rep_hbm,
                    idx_v, s_v, v4_v, r2_v, at_v, out_v, rep_v,
                    sem1, sem2, sem3, sem4, sem5):
    wid = lax.axis_index("s") * NC + lax.axis_index("c")
    base = wid * CHUNK
    ci = pltpu.async_copy(idx_hbm.at[pl.ds(base, CHUNK)], idx_v, sem1)
    cg = pltpu.async_copy(sg_hbm.at[pl.ds(base, CHUNK)], s_v, sem2)
    cv = pltpu.async_copy(vr_hbm.at[pl.ds(base, CHUNK)], v4_v, sem3)
    cr = pltpu.async_copy(vr_hbm.at[pl.ds(B + base, CHUNK)], r2_v, sem4)
    ca = pltpu.async_copy(vr_hbm.at[pl.ds(2 * B, LANES)], at_v, sem5)
    cg.wait()
    cv.wait()
    cr.wait()
    scale = jnp.float32((1.0 - RHO) * float(N) * float(N))
    rep_acc = jnp.zeros((LANES,), jnp.float32)
    for j in range(CHUNK // LANES):
        sl = pl.ds(j * LANES, LANES)
        s = s_v[sl]
        out_v[sl] = RHO * s + scale * v4_v[sl]
        rep_acc = rep_acc + r2_v[sl] / s
    ca.wait()
    inv_n2 = 1.0 / (float(N) * float(N) * float(LANES))
    w0 = jnp.where(wid == 0, jnp.float32(inv_n2), jnp.float32(0.0))
    rep_v[...] = rep_acc + at_v[...] * w0
    ci.wait()
    cs = pltpu.async_copy(out_v, s_ref.at[idx_v], sem1)
    cp = pltpu.async_copy(rep_v, rep_hbm.at[pl.ds(wid * LANES, LANES)], sem2)
    cs.wait()
    cp.wait()


def _dense_body(f_hbm, vr_hbm, f_v, vr_v, semf, semo):
    pltpu.async_copy(f_hbm, f_v, semf).wait()
    f = f_v[...]
    norm = jnp.maximum(jnp.sqrt(jnp.sum(f * f, axis=1, keepdims=True)), 1e-12)
    fn = f / norm
    an = fn[:B]
    bn = fn[B:]
    bro = pltpu.roll(bn, B - 1, 0)
    aro = pltpu.roll(an, B - 1, 0)
    d2aa = jnp.sum((an - bn + EPS) ** 2, axis=1, keepdims=True)
    d2bb = jnp.sum((bn - an + EPS) ** 2, axis=1, keepdims=True)
    d2ra = jnp.sum((an - bro + EPS) ** 2, axis=1, keepdims=True)
    d2rb = jnp.sum((bn - aro + EPS) ** 2, axis=1, keepdims=True)
    inv2t2 = 1.0 / (2.0 * TEMP * TEMP)
    qaa = jnp.exp(-inv2t2 * d2aa)
    qab = jnp.exp(-inv2t2 * d2bb)
    qra = jnp.exp(-inv2t2 * d2ra)
    qrb = jnp.exp(-inv2t2 * d2rb)
    v4 = (ALPHA * 0.5) * (qaa + qab) + ((1.0 - ALPHA) * 0.5) * (qra + qrb)
    r2 = qra + qrb
    attr = inv2t2 * jnp.sum(d2aa + d2bb)
    vr_v[0:32, :] = v4.reshape(32, 128)
    vr_v[32:64, :] = r2.reshape(32, 128)
    vr_v[64:72, :] = jnp.full((8, 128), attr, jnp.float32)
    pltpu.async_copy(vr_v, vr_hbm, semo).wait()


@functools.cache
def _build():
    sc_mesh = plsc.VectorSubcoreMesh(
        core_axis_name="c", subcore_axis_name="s", num_cores=NC, num_subcores=NS
    )
    sc_gather = pl.kernel(
        _sc_gather_body,
        out_type=jax.ShapeDtypeStruct((B,), jnp.float32),
        mesh=sc_mesh,
        scratch_types=[
            pltpu.VMEM((CHUNK,), jnp.int32),
            pltpu.VMEM((CHUNK,), jnp.float32),
            pltpu.VMEM((CPCH,), jnp.float32),
            pltpu.VMEM((TAIL,), jnp.float32),
            pltpu.SemaphoreType.DMA,
            pltpu.SemaphoreType.DMA,
            pltpu.SemaphoreType.DMA,
        ],
    )
    sc_update = pl.kernel(
        _sc_update_body,
        out_type=jax.ShapeDtypeStruct((NW * LANES,), jnp.float32),
        mesh=sc_mesh,
        scratch_types=[
            pltpu.VMEM((CHUNK,), jnp.int32),
            pltpu.VMEM((CHUNK,), jnp.float32),
            pltpu.VMEM((CHUNK,), jnp.float32),
            pltpu.VMEM((CHUNK,), jnp.float32),
            pltpu.VMEM((LANES,), jnp.float32),
            pltpu.VMEM((CHUNK,), jnp.float32),
            pltpu.VMEM((LANES,), jnp.float32),
            pltpu.SemaphoreType.DMA,
            pltpu.SemaphoreType.DMA,
            pltpu.SemaphoreType.DMA,
            pltpu.SemaphoreType.DMA,
            pltpu.SemaphoreType.DMA,
        ],
    )
    tc_mesh = pltpu.create_tensorcore_mesh("t", num_cores=1)
    dense = pl.kernel(
        _dense_body,
        out_type=jax.ShapeDtypeStruct((72, 128), jnp.float32),
        mesh=tc_mesh,
        scratch_types=[
            pltpu.VMEM((2 * B, 128), jnp.float32),
            pltpu.VMEM((72, 128), jnp.float32),
            pltpu.SemaphoreType.DMA,
            pltpu.SemaphoreType.DMA,
        ],
    )
    return sc_gather, sc_update, dense


def kernel(feats, s_inv, feats_idx):
    sc_gather, sc_update, dense = _build()
    idx = feats_idx.astype(jnp.int32)
    s_ref = _empty_ref(jax.ShapeDtypeStruct((N,), jnp.float32))
    s_gather = sc_gather(s_inv, idx, s_ref)
    vr = dense(feats)
    flat = vr.reshape(72 * 128)
    rep = sc_update(s_ref, idx, s_gather, flat)
    new_s_inv = s_ref[...]
    n2 = jnp.float32(N) * jnp.float32(N)
    loss = 0.5 * n2 * jnp.sum(rep) / jnp.float32(B)
    return loss, new_s_inv
